# Initial kernel scaffold; baseline (speedup 1.0000x reference)
import jax
import jax.numpy as jnp
from jax.experimental import pallas as pl


def kernel(layer_input, router_w, w_in, w_out):
    raise NotImplementedError("write your pallas kernel here")


# speedup vs baseline: 3.4152x; 3.4152x over previous
import functools

import jax
import jax.numpy as jnp
from jax import lax
from jax.experimental import pallas as pl
from jax.experimental.pallas import tpu as pltpu
from jax.experimental.pallas import tpu_sc as plsc

N = 2048
D = 768
E = 64
F = 512
K = 2
T = 64
NTMAX = 128
NPMAX = NTMAX * T
NW = 32

_SC_MESH = plsc.VectorSubcoreMesh(
    core_axis_name="c", subcore_axis_name="s", num_cores=2, num_subcores=16)


def _router_body(x_ref, rw_ref, eid_ref, g_ref):
    x = x_ref[...]
    rw = rw_ref[...]
    logits = lax.dot_general(x, rw, (((1,), (1,)), ((), ())),
                             preferred_element_type=jnp.float32)
    iota = lax.broadcasted_iota(jnp.int32, logits.shape, 1)
    m1 = jnp.max(logits, axis=1, keepdims=True)
    i1 = jnp.min(jnp.where(logits == m1, iota, E), axis=1)
    masked = jnp.where(iota == i1[:, None], -jnp.inf, logits)
    m2 = jnp.max(masked, axis=1, keepdims=True)
    i2 = jnp.min(jnp.where(masked == m2, iota, E), axis=1)
    s = jnp.exp(m2[:, 0] - m1[:, 0])
    g1 = 1.0 / (1.0 + s)
    g2 = s / (1.0 + s)
    eid_ref[...] = jnp.stack([i1, i2], axis=1)
    g_ref[...] = jnp.stack([g1, g2], axis=1)


def _router(x2d, router_w):
    return pl.pallas_call(
        _router_body,
        out_shape=(jax.ShapeDtypeStruct((N, K), jnp.int32),
                   jax.ShapeDtypeStruct((N, K), jnp.float32)),
    )(x2d, router_w)


@functools.partial(
    pl.kernel,
    out_type=jax.ShapeDtypeStruct((NPMAX, D), jnp.float32),
    mesh=_SC_MESH,
    scratch_types=[
        pltpu.VMEM((K * N // NW,), jnp.int32),
        pltpu.VMEM((K * N // NW,), jnp.int32),
        pltpu.VMEM((K * N // NW, D), jnp.float32),
        pltpu.SemaphoreType.DMA,
        pltpu.SemaphoreType.DMA,
    ],
)
def _dispatch(x_hbm, src_hbm, dst_hbm, xpad_hbm, src_v, dst_v, rows_v,
              sem_g, sem_s):
    per_w = K * N // NW
    wid = lax.axis_index("s") * 2 + lax.axis_index("c")
    base = wid * per_w
    pltpu.sync_copy(src_hbm.at[pl.ds(base, per_w)], src_v)
    pltpu.sync_copy(dst_hbm.at[pl.ds(base, per_w)], dst_v)
    pltpu.async_copy(x_hbm.at[src_v], rows_v, sem_g).wait()
    pltpu.async_copy(rows_v, xpad_hbm.at[dst_v], sem_s).wait()


def _mlp_body(te_ref, rb_ref, x_ref, wi_ref, wo_ref, gp_ref, y_ref):
    del te_ref, rb_ref
    x = x_ref[...]
    wi = wi_ref[0]
    h = lax.dot_general(x, wi, (((1,), (1,)), ((), ())),
                        preferred_element_type=jnp.float32)
    g = h[:, :F]
    u = h[:, F:]
    act = g * jax.nn.sigmoid(g) * u
    wo = wo_ref[0]
    y = lax.dot_general(act, wo, (((1,), (1,)), ((), ())),
                        preferred_element_type=jnp.float32)
    gate = gp_ref[0, 0, :]
    y_ref[...] = y * gate[:, None]


def _mlp(tile_expert, row_block, x_padded, w_in, w_out, gates_pad3):
    grid_spec = pltpu.PrefetchScalarGridSpec(
        num_scalar_prefetch=2,
        grid=(NTMAX,),
        in_specs=[
            pl.BlockSpec((T, D), lambda t, te, rb: (rb[t], 0)),
            pl.BlockSpec((1, 2 * F, D), lambda t, te, rb: (te[t], 0, 0)),
            pl.BlockSpec((1, D, F), lambda t, te, rb: (te[t], 0, 0)),
            pl.BlockSpec((1, 1, T), lambda t, te, rb: (rb[t], 0, 0)),
        ],
        out_specs=pl.BlockSpec((T, D), lambda t, te, rb: (rb[t], 0)),
    )
    return pl.pallas_call(
        _mlp_body,
        grid_spec=grid_spec,
        out_shape=jax.ShapeDtypeStruct((NPMAX, D), jnp.float32),
    )(tile_expert, row_block, x_padded, w_in, w_out, gates_pad3)


@functools.partial(
    pl.kernel,
    out_type=jax.ShapeDtypeStruct((N, D), jnp.float32),
    mesh=_SC_MESH,
    scratch_types=[
        pltpu.VMEM((N // NW,), jnp.int32),
        pltpu.VMEM((N // NW,), jnp.int32),
        pltpu.VMEM((N // NW, D), jnp.float32),
        pltpu.VMEM((N // NW, D), jnp.float32),
        pltpu.SemaphoreType.DMA,
        pltpu.SemaphoreType.DMA,
    ],
)
def _combine(y_hbm, posa_hbm, posb_hbm, out_hbm, ia_v, ib_v, a_v, b_v,
             sem_a, sem_b):
    per_w = N // NW
    wid = lax.axis_index("s") * 2 + lax.axis_index("c")
    base = wid * per_w
    pltpu.sync_copy(posa_hbm.at[pl.ds(base, per_w)], ia_v)
    pltpu.sync_copy(posb_hbm.at[pl.ds(base, per_w)], ib_v)
    ca = pltpu.async_copy(y_hbm.at[ia_v], a_v, sem_a)
    cb = pltpu.async_copy(y_hbm.at[ib_v], b_v, sem_b)
    ca.wait()
    cb.wait()

    def row(r, carry):
        for c in range(D // 16):
            sl = pl.ds(c * 16, 16)
            a_v[r, sl] = a_v[r, sl] + b_v[r, sl]
        return carry

    lax.fori_loop(0, per_w, row, 0)
    pltpu.sync_copy(a_v, out_hbm.at[pl.ds(base, per_w)])


def kernel(layer_input, router_w, w_in, w_out):
    B, S, _ = layer_input.shape
    x2d = layer_input.reshape(N, D)

    eids, gates = _router(x2d, router_w)

    eflat = eids.reshape(-1)
    gflat = gates.reshape(-1)
    order = jnp.argsort(eflat, stable=True).astype(jnp.int32)
    e_sorted = jnp.take(eflat, order)
    counts = jnp.bincount(eflat, length=E).astype(jnp.int32)
    ntiles = (counts + (T - 1)) // T
    cum_nt = jnp.cumsum(ntiles)
    nt_used = cum_nt[E - 1]
    tile_start = cum_nt - ntiles
    off = jnp.cumsum(counts) - counts
    pad_off = tile_start * T
    j = jnp.arange(K * N, dtype=jnp.int32)
    dst = jnp.take(pad_off, e_sorted) + j - jnp.take(off, e_sorted)
    src = (order // K).astype(jnp.int32)

    tt = jnp.minimum(jnp.arange(NTMAX, dtype=jnp.int32), nt_used - 1)
    tile_expert = jnp.searchsorted(cum_nt, tt, side="right").astype(jnp.int32)
    row_block = tt

    gates_sorted = jnp.take(gflat, order)
    gates_pad = jnp.zeros((NPMAX,), jnp.float32).at[dst].set(gates_sorted)
    pos_pair = jnp.zeros((K * N,), jnp.int32).at[order].set(dst)
    posa = pos_pair[0::2]
    posb = pos_pair[1::2]

    x_padded = _dispatch(x2d, src, dst)
    y_padded = _mlp(tile_expert, row_block, x_padded, w_in, w_out,
                    gates_pad.reshape(NTMAX, 1, T))
    out2d = _combine(y_padded, posa, posb)
    return out2d.reshape(B, S, D)


# speedup vs baseline: 4.9529x; 1.4503x over previous
import functools

import jax
import jax.numpy as jnp
from jax import lax
from jax.experimental import pallas as pl
from jax.experimental.pallas import tpu as pltpu
from jax.experimental.pallas import tpu_sc as plsc

N = 2048
D = 768
E = 64
F = 512
K = 2
T = 64
NTMAX = 128
NPMAX = NTMAX * T
NW = 32

_SC_MESH = plsc.VectorSubcoreMesh(
    core_axis_name="c", subcore_axis_name="s", num_cores=2, num_subcores=16)


def _excl_cumsum_rows(a):
    n = a.shape[0]
    acc = a
    k = 1
    while k < n:
        shifted = jnp.concatenate(
            [jnp.zeros((k,) + a.shape[1:], a.dtype), acc[:-k]], axis=0)
        acc = acc + shifted
        k *= 2
    return acc - a, acc[-1:]


def _cumsum_lanes(a):
    n = a.shape[1]
    acc = a
    k = 1
    while k < n:
        shifted = jnp.concatenate(
            [jnp.zeros((1, k), a.dtype), acc[:, :-k]], axis=1)
        acc = acc + shifted
        k *= 2
    return acc


def _router_body(x_ref, rw_ref, dst_ref, g_ref, cnt_ref):
    x = x_ref[...]
    rw = rw_ref[...]
    logits = lax.dot_general(x, rw, (((1,), (1,)), ((), ())),
                             preferred_element_type=jnp.float32)
    iota = lax.broadcasted_iota(jnp.int32, logits.shape, 1)
    m1 = jnp.max(logits, axis=1, keepdims=True)
    i1 = jnp.min(jnp.where(logits == m1, iota, E), axis=1)
    masked = jnp.where(iota == i1[:, None], -jnp.inf, logits)
    m2 = jnp.max(masked, axis=1, keepdims=True)
    i2 = jnp.min(jnp.where(masked == m2, iota, E), axis=1)
    s = jnp.exp(m2[:, 0] - m1[:, 0])
    g1 = 1.0 / (1.0 + s)
    g2 = s / (1.0 + s)
    oh1 = (iota == i1[:, None]).astype(jnp.int32)
    oh2 = (iota == i2[:, None]).astype(jnp.int32)
    excl, counts = _excl_cumsum_rows(oh1 + oh2)
    cum_nt = _cumsum_lanes((counts + T - 1) // T)
    pad_off = (cum_nt * T) - ((counts + T - 1) // T) * T
    rank1 = jnp.sum(excl * oh1, axis=1)
    rank2 = jnp.sum(excl * oh2, axis=1)
    po1 = jnp.sum(pad_off * oh1, axis=1)
    po2 = jnp.sum(pad_off * oh2, axis=1)
    dst_ref[...] = jnp.stack([po1 + rank1, po2 + rank2], axis=1)
    g_ref[...] = jnp.stack([g1, g2], axis=1)
    cnt_ref[...] = cum_nt


def _router(x2d, router_w):
    return pl.pallas_call(
        _router_body,
        out_shape=(jax.ShapeDtypeStruct((N, K), jnp.int32),
                   jax.ShapeDtypeStruct((N, K), jnp.float32),
                   jax.ShapeDtypeStruct((1, E), jnp.int32)),
    )(x2d, router_w)


@functools.partial(
    pl.kernel,
    out_type=jax.ShapeDtypeStruct((NPMAX, D), jnp.float32),
    mesh=_SC_MESH,
    scratch_types=[
        pltpu.VMEM((K * N // NW,), jnp.int32),
        pltpu.VMEM((K * N // NW,), jnp.int32),
        pltpu.VMEM((K * N // NW, D), jnp.float32),
        pltpu.SemaphoreType.DMA,
        pltpu.SemaphoreType.DMA,
    ],
)
def _dispatch(x_hbm, src_hbm, dst_hbm, xpad_hbm, src_v, dst_v, rows_v,
              sem_g, sem_s):
    per_w = K * N // NW
    wid = lax.axis_index("s") * 2 + lax.axis_index("c")
    base = wid * per_w
    pltpu.sync_copy(src_hbm.at[pl.ds(base, per_w)], src_v)
    pltpu.sync_copy(dst_hbm.at[pl.ds(base, per_w)], dst_v)
    pltpu.async_copy(x_hbm.at[src_v], rows_v, sem_g).wait()
    pltpu.async_copy(rows_v, xpad_hbm.at[dst_v], sem_s).wait()


def _mlp_body(te_ref, rb_ref, x_ref, wi_ref, wo_ref, gp_ref, y_ref):
    del te_ref, rb_ref
    x = x_ref[...]
    wi = wi_ref[0]
    h = lax.dot_general(x, wi, (((1,), (1,)), ((), ())),
                        preferred_element_type=jnp.float32)
    g = h[:, :F]
    u = h[:, F:]
    act = g * jax.nn.sigmoid(g) * u
    wo = wo_ref[0]
    y = lax.dot_general(act, wo, (((1,), (1,)), ((), ())),
                        preferred_element_type=jnp.float32)
    gate = gp_ref[0, 0, :]
    y_ref[...] = y * gate[:, None]


def _mlp(tile_expert, row_block, x_padded, w_in, w_out, gates_pad3):
    grid_spec = pltpu.PrefetchScalarGridSpec(
        num_scalar_prefetch=2,
        grid=(NTMAX,),
        in_specs=[
            pl.BlockSpec((T, D), lambda t, te, rb: (rb[t], 0)),
            pl.BlockSpec((1, 2 * F, D), lambda t, te, rb: (te[t], 0, 0)),
            pl.BlockSpec((1, D, F), lambda t, te, rb: (te[t], 0, 0)),
            pl.BlockSpec((1, 1, T), lambda t, te, rb: (rb[t], 0, 0)),
        ],
        out_specs=pl.BlockSpec((T, D), lambda t, te, rb: (rb[t], 0)),
    )
    return pl.pallas_call(
        _mlp_body,
        grid_spec=grid_spec,
        out_shape=jax.ShapeDtypeStruct((NPMAX, D), jnp.float32),
    )(tile_expert, row_block, x_padded, w_in, w_out, gates_pad3)


@functools.partial(
    pl.kernel,
    out_type=jax.ShapeDtypeStruct((N, D), jnp.float32),
    mesh=_SC_MESH,
    scratch_types=[
        pltpu.VMEM((N // NW,), jnp.int32),
        pltpu.VMEM((N // NW,), jnp.int32),
        pltpu.VMEM((N // NW, D), jnp.float32),
        pltpu.VMEM((N // NW, D), jnp.float32),
        pltpu.SemaphoreType.DMA,
        pltpu.SemaphoreType.DMA,
    ],
)
def _combine(y_hbm, posa_hbm, posb_hbm, out_hbm, ia_v, ib_v, a_v, b_v,
             sem_a, sem_b):
    per_w = N // NW
    wid = lax.axis_index("s") * 2 + lax.axis_index("c")
    base = wid * per_w
    pltpu.sync_copy(posa_hbm.at[pl.ds(base, per_w)], ia_v)
    pltpu.sync_copy(posb_hbm.at[pl.ds(base, per_w)], ib_v)
    ca = pltpu.async_copy(y_hbm.at[ia_v], a_v, sem_a)
    cb = pltpu.async_copy(y_hbm.at[ib_v], b_v, sem_b)
    ca.wait()
    cb.wait()

    def row(r, carry):
        for c in range(D // 16):
            sl = pl.ds(c * 16, 16)
            a_v[r, sl] = a_v[r, sl] + b_v[r, sl]
        return carry

    lax.fori_loop(0, per_w, row, 0)
    pltpu.sync_copy(a_v, out_hbm.at[pl.ds(base, per_w)])


def kernel(layer_input, router_w, w_in, w_out):
    B, S, _ = layer_input.shape
    x2d = layer_input.reshape(N, D)

    dst2, gates, cum_nt = _router(x2d, router_w)

    dst = dst2.reshape(K * N)
    src = jnp.arange(K * N, dtype=jnp.int32) // K
    nt_used = cum_nt[0, E - 1]
    tt = jnp.minimum(jnp.arange(NTMAX, dtype=jnp.int32), nt_used - 1)
    tile_expert = jnp.searchsorted(cum_nt[0], tt, side="right").astype(jnp.int32)
    row_block = tt
    gates_pad = jnp.zeros((NPMAX,), jnp.float32).at[dst].set(gates.reshape(-1))
    posa = dst2[:, 0]
    posb = dst2[:, 1]

    x_padded = _dispatch(x2d, src, dst)
    y_padded = _mlp(tile_expert, row_block, x_padded, w_in, w_out,
                    gates_pad.reshape(NTMAX, 1, T))
    out2d = _combine(y_padded, posa, posb)
    return out2d.reshape(B, S, D)
